# baseline (speedup 1.0000x reference)
import functools

import jax
import jax.numpy as jnp
from jax.experimental import pallas as pl
from jax.experimental.pallas import tpu as pltpu

_SMOOTHING = 0.1
_LOG2E = 1.4426950408889634
_LN2 = 0.6931471805599453
_NSTREAM = 4
_BLOCK_ROWS = 256
_SLICE = 8


def _bce_body(*refs, nsteps, inv_n, off_value, on_minus_off):
    x_refs = refs[:_NSTREAM]
    tgt_ref = refs[_NSTREAM]
    o_ref = refs[_NSTREAM + 1]
    acc_ref = refs[_NSTREAM + 2]
    i = pl.program_id(0)

    @pl.when(i == 0)
    def _init():
        acc_ref[...] = jnp.zeros_like(acc_ref)

    c = x_refs[0].shape[1]
    k2 = (0.5 - off_value) / _LN2
    k_on = k2 - on_minus_off / _LN2
    col = jax.lax.broadcasted_iota(jnp.int32, (1, c), 1)

    acc_m = acc_ref[0, :, :]
    acc_a = acc_ref[1, :, :]
    for k, x_ref in enumerate(x_refs):
        row0 = (i + k * nsteps) * _BLOCK_ROWS
        for r in range(_BLOCK_ROWS // _SLICE):
            xs = x_ref[pl.ds(_SLICE * r, _SLICE), :]
            tgt8 = tgt_ref[pl.ds(row0 + _SLICE * r, _SLICE), :]
            a = jnp.abs(xs)
            l = jnp.log2(1.0 + jnp.exp2(a * (-_LOG2E)))
            w = jnp.where(col == tgt8, k_on, k2)
            acc_m = acc_m + (l + xs * w)
            acc_a = acc_a + a
    acc_ref[0, :, :] = acc_m
    acc_ref[1, :, :] = acc_a

    @pl.when(i == nsteps - 1)
    def _finish():
        total = jnp.sum(acc_m) + 0.5 / _LN2 * jnp.sum(acc_a)
        o_ref[...] = jnp.full((1, 1), _LN2 * inv_n) * total


def kernel(x, target):
    b, c = x.shape
    off_value = _SMOOTHING / c
    tgt = target.reshape(b, 1).astype(jnp.int32)

    nsteps = b // (_NSTREAM * _BLOCK_ROWS)

    x_specs = [
        pl.BlockSpec((_BLOCK_ROWS, c), lambda i, k=k, n=nsteps: (i + k * n, 0))
        for k in range(_NSTREAM)
    ]
    t_spec = pl.BlockSpec((b, 1), lambda i: (0, 0))

    out = pl.pallas_call(
        functools.partial(
            _bce_body,
            nsteps=nsteps,
            inv_n=1.0 / (b * c),
            off_value=float(off_value),
            on_minus_off=float(1.0 - _SMOOTHING),
        ),
        grid=(nsteps,),
        in_specs=x_specs + [t_spec],
        out_specs=pl.BlockSpec((1, 1), lambda i: (0, 0)),
        out_shape=jax.ShapeDtypeStruct((1, 1), jnp.float32),
        scratch_shapes=[pltpu.VMEM((2, _SLICE, c), jnp.float32)],
    )(*([x] * _NSTREAM + [tgt]))
    return out[0, 0]
